# Initial kernel scaffold; baseline (speedup 1.0000x reference)
import jax
import jax.numpy as jnp
from jax.experimental import pallas as pl


def kernel(gt_boxes, spatial_indices):
    raise NotImplementedError("write your pallas kernel here")


# speedup vs baseline: 3.5973x; 3.5973x over previous
import functools

import jax
import jax.numpy as jnp
from jax import lax
from jax.experimental import pallas as pl
from jax.experimental.pallas import tpu as pltpu
from jax.experimental.pallas import tpu_sc as plsc

NUM_CLASSES = 3
K = 5
M = 500
MP = 512
N = 65536
R = 8
FLAT = MP * K


def _topk_body(gt_ref, vp_ref, cd_ref, mask_ref, inds_ref, hmf_ref, rb_ref,
               dist_ref):
    f32 = jnp.float32
    i = pl.program_id(0)
    g = gt_ref[...]
    x, y, z = g[:, 0:1], g[:, 1:2], g[:, 2:3]
    sx, sy, sz = g[:, 3:4], g[:, 4:5], g[:, 5:6]
    head, clsf = g[:, 6:7], g[:, 7:8]

    valid_b = ((sx > 0) & (sy > 0) & (sz > 0)
               & (x >= f32(-75.2)) & (y >= f32(-75.2))
               & (x < f32(75.2)) & (y < f32(75.2)))
    valid = valid_b.astype(f32)

    cx = jnp.clip((x - f32(-75.2)) / f32(0.1) / f32(4.0), f32(0.0), f32(375.5))
    cy = jnp.clip((y - f32(-75.2)) / f32(0.1) / f32(4.0), f32(0.0), f32(375.5))
    dxw = sx / f32(0.1) / f32(4.0)
    dyw = sy / f32(0.1) / f32(4.0)
    radius = jnp.sqrt((dxw / 2.0) ** 2 + (dyw / 2.0) ** 2)

    vxh = vp_ref[0:1, :]
    vyh = vp_ref[1:2, :]
    pk = vp_ref[2:3, :]
    dist_ref[...] = jnp.abs(vxh - cx) + jnp.abs(vyh - cy)

    iota = lax.broadcasted_iota(jnp.int32, (R, N), 1)
    vals, idxs, gpks = [], [], []
    for _ in range(K):
        d = dist_ref[...]
        m = jnp.min(d, axis=1, keepdims=True)
        idx = jnp.min(jnp.where(d == m, iota, N), axis=1, keepdims=True)
        win = iota == idx
        gpk = jnp.sum(jnp.where(win, pk, f32(0.0)), axis=1, keepdims=True)
        dist_ref[...] = jnp.where(win, f32(jnp.inf), d)
        vals.append(m)
        idxs.append(idx)
        gpks.append(gpk)
    valsm = jnp.concatenate(vals, axis=1)
    indsm = jnp.concatenate(idxs, axis=1)
    gpkm = jnp.concatenate(gpks, axis=1)

    cd_ref[...] = valsm * valid

    rio = lax.broadcasted_iota(jnp.int32, (R, K), 0)
    grow = rio + i * R
    base_mask = (valsm <= radius).astype(f32)
    mask_ref[...] = jnp.where(grow == 0, f32(1.0), base_mask) * valid

    inds_ref[...] = indsm * valid.astype(jnp.int32)

    cls_id = jnp.clip(clsf - 1.0, 0.0, float(NUM_CLASSES - 1)).astype(jnp.int32)
    hmf_ref[...] = jnp.where(valid_b, cls_id * N + indsm, NUM_CLASSES * N)

    gx = jnp.floor(gpkm * f32(1.0 / 512.0))
    gy = gpkm - gx * f32(512.0)
    offx = (cx - gx) - f32(0.5)
    offy = (cy - gy) - f32(0.5)
    lx, ly, lz = jnp.log(sx), jnp.log(sy), jnp.log(sz)
    ch, sh = jnp.cos(head), jnp.sin(head)
    cols = []
    for k in range(K):
        cols.extend([offx[:, k:k + 1], offy[:, k:k + 1], z, lx, ly, lz, ch, sh])
    rb_ref[...] = jnp.concatenate(cols, axis=1) * valid


def _run_topk(gt_pad, vp):
    out_shapes = [
        jax.ShapeDtypeStruct((MP, K), jnp.float32),
        jax.ShapeDtypeStruct((MP, K), jnp.float32),
        jax.ShapeDtypeStruct((MP, K), jnp.int32),
        jax.ShapeDtypeStruct((MP, K), jnp.int32),
        jax.ShapeDtypeStruct((MP, K * 8), jnp.float32),
    ]
    small = lambda w, d: pl.BlockSpec((R, w), lambda i: (i, 0))
    return pl.pallas_call(
        _topk_body,
        grid=(MP // R,),
        in_specs=[
            pl.BlockSpec((R, 8), lambda i: (i, 0)),
            pl.BlockSpec((8, N), lambda i: (0, 0)),
        ],
        out_specs=[
            small(K, jnp.float32),
            small(K, jnp.float32),
            small(K, jnp.int32),
            small(K, jnp.int32),
            small(K * 8, jnp.float32),
        ],
        out_shape=out_shapes,
        scratch_shapes=[pltpu.VMEM((R, N), jnp.float32)],
    )(gt_pad, vp)


def _hm_body(hmf_hbm, zero_hbm, out_hbm, idx_v, row_v):
    cid = lax.axis_index("c")
    sid = lax.axis_index("s")

    @pl.when((cid == 0) & (sid < NUM_CLASSES))
    def _():
        pltpu.sync_copy(zero_hbm.at[sid], row_v)
        pltpu.sync_copy(hmf_hbm, idx_v)
        ones = jnp.full((16,), 1.0, jnp.float32)
        base = sid * N

        def body(j, carry):
            ids = idx_v[pl.ds(j * 16, 16)]
            msk = (ids >= base) & (ids < base + N)
            loc = jnp.where(msk, ids - base, 0)
            plsc.store_scatter(row_v, [loc], ones, mask=msk)
            return carry

        lax.fori_loop(0, FLAT // 16, body, 0)
        pltpu.sync_copy(row_v, out_hbm.at[sid])


@functools.cache
def _hm_scatter():
    mesh = plsc.VectorSubcoreMesh(core_axis_name="c", subcore_axis_name="s")
    return pl.kernel(
        _hm_body,
        out_type=jax.ShapeDtypeStruct((NUM_CLASSES, N), jnp.float32),
        mesh=mesh,
        scratch_types=[
            pltpu.VMEM((FLAT,), jnp.int32),
            pltpu.VMEM((N,), jnp.float32),
        ],
        compiler_params=pltpu.CompilerParams(needs_layout_passes=False),
    )


def kernel(gt_boxes, spatial_indices):
    vox = spatial_indices.astype(jnp.float32)
    vxh = vox[:, 0] + 0.5
    vyh = vox[:, 1] + 0.5
    pk = vox[:, 0] * 512.0 + vox[:, 1]
    zero_row = jnp.zeros((N,), jnp.float32)
    vp = jnp.stack([vxh, vyh, pk, zero_row, zero_row, zero_row, zero_row,
                    zero_row], axis=0)
    gt_pad = jnp.zeros((MP, 8), jnp.float32).at[:M].set(gt_boxes)

    cd, mask, inds, hmf, rb2d = _run_topk(gt_pad, vp)
    heatmap = _hm_scatter()(hmf.reshape(FLAT),
                            jnp.zeros((NUM_CLASSES, N), jnp.float32))
    ret_boxes = rb2d[:M].reshape(M, K, 8)
    return heatmap, ret_boxes, cd[:M], inds[:M], mask[:M]
